# Initial kernel scaffold; baseline (speedup 1.0000x reference)
import jax
import jax.numpy as jnp
from jax.experimental import pallas as pl


def kernel(x, edge_index, edge_weight, W1, b1, W2, b2):
    raise NotImplementedError("write your pallas kernel here")


# speedup vs baseline: 4.0156x; 4.0156x over previous
import functools

import jax
import jax.numpy as jnp
from jax import lax
from jax.experimental import pallas as pl
from jax.experimental.pallas import tpu as pltpu
from jax.experimental.pallas import tpu_sc as plsc

N = 10000
E = 320000
NC = 2
NS = 16
EPW = E // (NC * NS)
EB = 80
NB = EPW // EB
NPAD = 10240
RPT = NPAD // NS
ZR = 128


_GDN = lax.GatherDimensionNumbers(
    offset_dims=(), collapsed_slice_dims=(0,), start_index_map=(0,))


def _lane_broadcast(vec, lane):
    idx = jnp.full((16, 1), lane, jnp.int32)
    return lax.gather(vec, idx, _GDN, slice_sizes=(1,),
                      mode=lax.GatherScatterMode.PROMISE_IN_BOUNDS)


def _make_spmm(W):
    FC = W // 16

    mesh = plsc.VectorSubcoreMesh(core_axis_name="c", subcore_axis_name="s")

    @functools.partial(
        pl.kernel,
        out_type=jax.ShapeDtypeStruct((NC, NPAD, W), jnp.float32),
        mesh=mesh,
        scratch_types=[
            pltpu.VMEM((EB,), jnp.int32),
            pltpu.VMEM((EB,), jnp.int32),
            pltpu.VMEM((EB,), jnp.float32),
            pltpu.VMEM((EB, W), jnp.float32),
            pltpu.VMEM((ZR, W), jnp.float32),
            pltpu.VMEM_SHARED((NPAD, W), jnp.float32),
            pltpu.SemaphoreType.DMA,
        ],
    )
    def spmm(h_hbm, src_hbm, dst_hbm, ew_hbm, out_hbm,
             srcv, dstv, wv, rows, zbuf, acc, sem):
        cid = lax.axis_index("c")
        sid = lax.axis_index("s")

        zvec = jnp.zeros((16,), jnp.float32)

        def zrow(r, carry):
            for f in range(FC):
                zbuf[r, pl.ds(f * 16, 16)] = zvec
            return carry

        lax.fori_loop(0, ZR, zrow, 0)
        for k in range(RPT // ZR):
            pltpu.sync_copy(zbuf, acc.at[pl.ds(sid * RPT + k * ZR, ZR)])
        plsc.subcore_barrier()

        ebase = (cid * NS + sid) * EPW

        def eblock(j, carry):
            base = ebase + j * EB
            pltpu.sync_copy(src_hbm.at[pl.ds(base, EB)], srcv)
            pltpu.sync_copy(dst_hbm.at[pl.ds(base, EB)], dstv)
            pltpu.sync_copy(ew_hbm.at[pl.ds(base, EB)], wv)
            pltpu.async_copy(h_hbm.at[srcv], rows, sem).wait()

            def scale(g, c2):
                wg = wv[pl.ds(g * 16, 16)]
                for i in range(16):
                    ws = _lane_broadcast(wg, i)
                    e = g * 16 + i
                    for f in range(FC):
                        sl = pl.ds(f * 16, 16)
                        rows[e, sl] = rows[e, sl] * ws
                return c2

            lax.fori_loop(0, EB // 16, scale, 0)
            pltpu.sync_copy(rows, acc.at[dstv], add=True)
            return carry

        lax.fori_loop(0, NB, eblock, 0)
        plsc.subcore_barrier()

        for k in range(RPT // ZR):
            r0 = sid * RPT + k * ZR
            pltpu.sync_copy(acc.at[pl.ds(r0, ZR)], out_hbm.at[cid, pl.ds(r0, ZR)])

    return spmm


_spmm128 = _make_spmm(128)

_BM = 1000


def _mm_body(x_ref, w_ref, o_ref):
    o_ref[...] = jnp.dot(x_ref[...], w_ref[...],
                         preferred_element_type=jnp.float32)


def _matmul_tc(x, w):
    m, k = x.shape
    n = w.shape[1]
    return pl.pallas_call(
        _mm_body,
        grid=(m // _BM,),
        in_specs=[pl.BlockSpec((_BM, k), lambda i: (i, 0)),
                  pl.BlockSpec((k, n), lambda i: (0, 0))],
        out_specs=pl.BlockSpec((_BM, n), lambda i: (i, 0)),
        out_shape=jax.ShapeDtypeStruct((m, n), jnp.float32),
    )(x, w)


def _mid_body(p0_ref, p1_ref, b_ref, w_ref, o_ref):
    h = jnp.maximum(p0_ref[...] + p1_ref[...] + b_ref[...], 0.0)
    o_ref[...] = jnp.dot(h, w_ref[...], preferred_element_type=jnp.float32)


def _mid_tc(p0, p1, b1, w2):
    m, k = p0.shape
    n = w2.shape[1]
    return pl.pallas_call(
        _mid_body,
        grid=(m // _BM,),
        in_specs=[pl.BlockSpec((_BM, k), lambda i: (i, 0)),
                  pl.BlockSpec((_BM, k), lambda i: (i, 0)),
                  pl.BlockSpec((1, k), lambda i: (0, 0)),
                  pl.BlockSpec((k, n), lambda i: (0, 0))],
        out_specs=pl.BlockSpec((_BM, n), lambda i: (i, 0)),
        out_shape=jax.ShapeDtypeStruct((m, n), jnp.float32),
    )(p0, p1, b1, w2)


def _final_body(q0_ref, q1_ref, b_ref, o_ref):
    z = q0_ref[...] + q1_ref[...] + b_ref[...]
    z = z - jnp.max(z, axis=1, keepdims=True)
    o_ref[...] = z - jnp.log(jnp.sum(jnp.exp(z), axis=1, keepdims=True))


def _final_tc(q0, q1, b2):
    m, n = q0.shape
    return pl.pallas_call(
        _final_body,
        grid=(m // _BM,),
        in_specs=[pl.BlockSpec((_BM, n), lambda i: (i, 0)),
                  pl.BlockSpec((_BM, n), lambda i: (i, 0)),
                  pl.BlockSpec((1, n), lambda i: (0, 0))],
        out_specs=pl.BlockSpec((_BM, n), lambda i: (i, 0)),
        out_shape=jax.ShapeDtypeStruct((m, n), jnp.float32),
    )(q0, q1, b2)


def kernel(x, edge_index, edge_weight, W1, b1, W2, b2):
    src = edge_index[0]
    dst = edge_index[1]
    h1 = _matmul_tc(x, W1)
    p = _spmm128(h1, src, dst, edge_weight)
    w2p = jnp.concatenate([W2, jnp.zeros((W2.shape[0], 128 - W2.shape[1]),
                                         jnp.float32)], axis=1)
    h2 = _mid_tc(p[0, :N], p[1, :N], b1.reshape(1, -1), w2p)
    q = _spmm128(h2, src, dst, edge_weight)
    ncls = W2.shape[1]
    return _final_tc(q[0, :N, :ncls], q[1, :N, :ncls], b2.reshape(1, -1))
